# baseline (speedup 1.0000x reference)
import functools

import jax
import jax.numpy as jnp
from jax import lax
from jax.experimental import pallas as pl
from jax.experimental.pallas import tpu as pltpu
from jax.experimental.pallas import tpu_sc as plsc

B = 4096
F = 26
D = 16
V = 1000000
NW = 32
EPW = B // NW
RPW = EPW * F
PACK = 128 // D
OUT_ROWS = B * D // 128
ORPW = OUT_ROWS // NW
TPW = (V // 128) // NW
TAIL = V - TPW * NW * 128


@functools.cache
def _make_sc_detile():
    mesh = plsc.VectorSubcoreMesh(core_axis_name="c", subcore_axis_name="s")

    @functools.partial(
        pl.kernel,
        mesh=mesh,
        compiler_params=pltpu.CompilerParams(use_tc_tiling_on_sc=True,
                                             needs_layout_passes=False),
        out_type=jax.ShapeDtypeStruct((V * D,), jnp.float32),
        scratch_types=[
            pltpu.VMEM((D, 128), jnp.float32),
            pltpu.VMEM((D, 128), jnp.float32),
            pltpu.VMEM((128 * D,), jnp.float32),
            pltpu.VMEM((128 * D,), jnp.float32),
            pltpu.VMEM((8, D), jnp.float32),
            pltpu.SemaphoreType.DMA,
            pltpu.SemaphoreType.DMA,
            pltpu.SemaphoreType.DMA,
            pltpu.SemaphoreType.DMA,
        ],
    )
    def sc_detile(tt_hbm, tail_hbm, out_hbm, in0, in1, o0, o1, slab_v,
                  si0, si1, so0, so1):
        wid = lax.axis_index("s") * 2 + lax.axis_index("c")
        ins = (in0, in1)
        outs = (o0, o1)
        sis = (si0, si1)
        sos = (so0, so1)
        iota16 = lax.iota(jnp.int32, D)

        def col(i):
            return pl.multiple_of((wid * TPW + i) * 128, 128)

        def fire(i, k):
            pltpu.async_copy(tt_hbm.at[:, pl.ds(col(i), 128)], ins[k],
                             sis[k])

        def transpose(k):
            @plsc.parallel_loop(0, 128, unroll=16)
            def _(t):
                v0 = (t >> 4) * D
                r = t & 15
                cols = v0 + ((iota16 + r) & 15)
                val = plsc.load_gather(ins[k], [iota16, cols])
                plsc.store_scatter(outs[k], [cols * D + iota16], val)

        def put(i, k):
            pltpu.async_copy(outs[k], out_hbm.at[pl.ds(col(i) * D, 128 * D)],
                             sos[k])

        fire(0, 0)

        def body(j, carry):
            i = j * 2
            fire(i + 1, 1)
            pltpu.make_async_copy(tt_hbm.at[:, pl.ds(0, 128)], ins[0],
                                  sis[0]).wait()

            @pl.when(j > 0)
            def _():
                pltpu.make_async_copy(outs[0], out_hbm.at[pl.ds(0, 128 * D)],
                                      sos[0]).wait()
            transpose(0)
            put(i, 0)

            @pl.when(i + 2 < TPW)
            def _():
                fire(i + 2, 0)
            pltpu.make_async_copy(tt_hbm.at[:, pl.ds(0, 128)], ins[1],
                                  sis[1]).wait()

            @pl.when(j > 0)
            def _():
                pltpu.make_async_copy(outs[1], out_hbm.at[pl.ds(0, 128 * D)],
                                      sos[1]).wait()
            transpose(1)
            put(i + 1, 1)
            return carry

        lax.fori_loop(0, TPW // 2, body, 0)
        pltpu.make_async_copy(outs[0], out_hbm.at[pl.ds(0, 128 * D)],
                              sos[0]).wait()
        pltpu.make_async_copy(outs[1], out_hbm.at[pl.ds(0, 128 * D)],
                              sos[1]).wait()

        @pl.when(wid == 0)
        def _():
            base = TPW * NW * 128
            for c0 in [base, base + 128, base + 256, base + 384]:
                pltpu.sync_copy(tt_hbm.at[:, pl.ds(c0, 128)], in0)
                for v in range(128):
                    o0[pl.ds(v * D, D)] = plsc.load_gather(
                        in0, [iota16, jnp.full((D,), v, jnp.int32)])
                pltpu.sync_copy(o0, out_hbm.at[pl.ds(c0 * D, 128 * D)])
            c1 = base + 512
            for t in range(8):
                pltpu.sync_copy(tail_hbm.at[pl.ds(t * 8, 8)], slab_v)
                for r in range(8):
                    o0[pl.ds(r * D, D)] = slab_v[r]
                pltpu.sync_copy(o0.at[pl.ds(0, 8 * D)],
                                out_hbm.at[pl.ds((c1 + t * 8) * D, 8 * D)])

    return sc_detile


@functools.cache
def _make_sc_pool():
    mesh = plsc.VectorSubcoreMesh(core_axis_name="c", subcore_axis_name="s")

    @functools.partial(
        pl.kernel,
        mesh=mesh,
        compiler_params=pltpu.CompilerParams(use_tc_tiling_on_sc=False,
                                             needs_layout_passes=False),
        out_type=jax.ShapeDtypeStruct((OUT_ROWS, 128), jnp.float32),
        scratch_types=[
            pltpu.VMEM((EPW, F), jnp.int32),
            pltpu.VMEM((EPW, F), jnp.float32),
            pltpu.VMEM((RPW, D), jnp.float32),
            pltpu.VMEM((ORPW, 128), jnp.float32),
            pltpu.SemaphoreType.DMA,
        ],
    )
    def sc_pool(idx_hbm, fv_hbm, table_hbm, out_hbm,
                idx_v, fv_v, rows_v, bi_v, sem):
        wid = lax.axis_index("s") * 2 + lax.axis_index("c")
        base = wid * EPW
        pltpu.sync_copy(idx_hbm.at[pl.ds(base, EPW)], idx_v)
        pltpu.sync_copy(fv_hbm.at[pl.ds(base, EPW)], fv_v)

        def fire(e, carry):
            pltpu.async_copy(table_hbm.at[idx_v.at[e]],
                             rows_v.at[pl.ds(e * F, F)], sem)
            return carry
        lax.fori_loop(0, EPW, fire, 0)
        pltpu.make_async_copy(table_hbm.at[pl.ds(0, RPW)], rows_v, sem).wait()

        def e_body(e, carry):
            s = jnp.zeros((D,), jnp.float32)
            q = jnp.zeros((D,), jnp.float32)
            for f in range(F):
                row = rows_v[e * F + f]
                vb = plsc.load_gather(
                    fv_v, [jnp.full((D,), e, jnp.int32),
                           jnp.full((D,), f, jnp.int32)])
                ve = row * vb
                s = s + ve
                q = q + ve * ve
            bi = (s * s - q) * 0.5
            bi_v[e // PACK, pl.ds((e % PACK) * D, D)] = bi
            return carry

        lax.fori_loop(0, EPW, e_body, 0)
        pltpu.sync_copy(bi_v, out_hbm.at[pl.ds(wid * ORPW, ORPW)])

    return sc_pool


def _tc_mlp(bi_p, W1, b1, W2, b2, W3, b3):
    eye = jnp.eye(PACK, dtype=jnp.float32)
    W1p = jnp.kron(eye, W1)
    W2p = jnp.kron(eye, W2)
    W3p = jnp.kron(eye, W3)
    b1p = jnp.tile(b1, PACK).reshape(1, -1)
    b2p = jnp.tile(b2, PACK).reshape(1, -1)
    b3p = jnp.tile(b3, PACK).reshape(1, -1)

    def body(bi_ref, W1_ref, b1_ref, W2_ref, b2_ref, W3_ref, b3_ref, out_ref):
        h = jnp.maximum(
            jnp.dot(bi_ref[...], W1_ref[...], preferred_element_type=jnp.float32)
            + b1_ref[...], 0.0)
        h = jnp.maximum(
            jnp.dot(h, W2_ref[...], preferred_element_type=jnp.float32)
            + b2_ref[...], 0.0)
        o = jnp.dot(h, W3_ref[...], preferred_element_type=jnp.float32) + b3_ref[...]
        out_ref[...] = jax.nn.sigmoid(o)

    out = pl.pallas_call(
        body,
        out_shape=jax.ShapeDtypeStruct((OUT_ROWS, PACK), jnp.float32),
    )(bi_p, W1p, b1p, W2p, b2p, W3p, b3p)
    return out.reshape(B, 1)


def kernel(feat_index, feat_value, emb_table, W1, b1, W2, b2, W3, b3):
    fidx = feat_index.astype(jnp.int32)
    table_lin = _make_sc_detile()(emb_table.T, emb_table[V - 64:])\
        .reshape(V, D)
    bi_p = _make_sc_pool()(fidx, feat_value, table_lin)
    return _tc_mlp(bi_p, W1, b1, W2, b2, W3, b3)


# speedup vs baseline: 1.0156x; 1.0156x over previous
import functools

import jax
import jax.numpy as jnp
from jax import lax
from jax.experimental import pallas as pl
from jax.experimental.pallas import tpu as pltpu
from jax.experimental.pallas import tpu_sc as plsc

B = 4096
F = 26
D = 16
V = 1000000
NW = 32
EPW = B // NW
RPW = EPW * F
PACK = 128 // D
OUT_ROWS = B * D // 128
ORPW = OUT_ROWS // NW
TPW = (V // 128) // NW
TAIL = V - TPW * NW * 128


@functools.cache
def _make_sc_detile():
    mesh = plsc.VectorSubcoreMesh(core_axis_name="c", subcore_axis_name="s")

    @functools.partial(
        pl.kernel,
        mesh=mesh,
        compiler_params=pltpu.CompilerParams(use_tc_tiling_on_sc=True,
                                             needs_layout_passes=False),
        out_type=jax.ShapeDtypeStruct((V * D,), jnp.float32),
        scratch_types=[
            pltpu.VMEM((D, 128), jnp.float32),
            pltpu.VMEM((D, 128), jnp.float32),
            pltpu.VMEM((128 * D,), jnp.float32),
            pltpu.VMEM((128 * D,), jnp.float32),
            pltpu.VMEM((8, D), jnp.float32),
            pltpu.SemaphoreType.DMA,
            pltpu.SemaphoreType.DMA,
            pltpu.SemaphoreType.DMA,
            pltpu.SemaphoreType.DMA,
        ],
    )
    def sc_detile(tt_hbm, tail_hbm, out_hbm, in0, in1, o0, o1, slab_v,
                  si0, si1, so0, so1):
        wid = lax.axis_index("s") * 2 + lax.axis_index("c")
        ins = (in0, in1)
        outs = (o0, o1)
        sis = (si0, si1)
        sos = (so0, so1)
        iota16 = lax.iota(jnp.int32, D)

        def col(i):
            return pl.multiple_of((wid * TPW + i) * 128, 128)

        def fire(i, k):
            pltpu.async_copy(tt_hbm.at[:, pl.ds(col(i), 128)], ins[k],
                             sis[k])

        def transpose(k):
            @plsc.parallel_loop(0, 128, unroll=8)
            def _(t):
                v0 = (t >> 4) * D
                r = t & 15
                cols = v0 + ((iota16 + r) & 15)
                val = plsc.load_gather(ins[k], [iota16, cols])
                plsc.store_scatter(outs[k], [cols * D + iota16], val)

        def put(i, k):
            pltpu.async_copy(outs[k], out_hbm.at[pl.ds(col(i) * D, 128 * D)],
                             sos[k])

        fire(0, 0)

        def body(j, carry):
            i = j * 2
            fire(i + 1, 1)
            pltpu.make_async_copy(tt_hbm.at[:, pl.ds(0, 128)], ins[0],
                                  sis[0]).wait()

            @pl.when(j > 0)
            def _():
                pltpu.make_async_copy(outs[0], out_hbm.at[pl.ds(0, 128 * D)],
                                      sos[0]).wait()
            transpose(0)
            put(i, 0)

            @pl.when(i + 2 < TPW)
            def _():
                fire(i + 2, 0)
            pltpu.make_async_copy(tt_hbm.at[:, pl.ds(0, 128)], ins[1],
                                  sis[1]).wait()

            @pl.when(j > 0)
            def _():
                pltpu.make_async_copy(outs[1], out_hbm.at[pl.ds(0, 128 * D)],
                                      sos[1]).wait()
            transpose(1)
            put(i + 1, 1)
            return carry

        lax.fori_loop(0, TPW // 2, body, 0)
        pltpu.make_async_copy(outs[0], out_hbm.at[pl.ds(0, 128 * D)],
                              sos[0]).wait()
        pltpu.make_async_copy(outs[1], out_hbm.at[pl.ds(0, 128 * D)],
                              sos[1]).wait()

        @pl.when(wid == 0)
        def _():
            base = TPW * NW * 128
            for c0 in [base, base + 128, base + 256, base + 384]:
                pltpu.sync_copy(tt_hbm.at[:, pl.ds(c0, 128)], in0)
                for v in range(128):
                    o0[pl.ds(v * D, D)] = plsc.load_gather(
                        in0, [iota16, jnp.full((D,), v, jnp.int32)])
                pltpu.sync_copy(o0, out_hbm.at[pl.ds(c0 * D, 128 * D)])
            c1 = base + 512
            for t in range(8):
                pltpu.sync_copy(tail_hbm.at[pl.ds(t * 8, 8)], slab_v)
                for r in range(8):
                    o0[pl.ds(r * D, D)] = slab_v[r]
                pltpu.sync_copy(o0.at[pl.ds(0, 8 * D)],
                                out_hbm.at[pl.ds((c1 + t * 8) * D, 8 * D)])

    return sc_detile


@functools.cache
def _make_sc_pool():
    mesh = plsc.VectorSubcoreMesh(core_axis_name="c", subcore_axis_name="s")

    @functools.partial(
        pl.kernel,
        mesh=mesh,
        compiler_params=pltpu.CompilerParams(use_tc_tiling_on_sc=False,
                                             needs_layout_passes=False),
        out_type=jax.ShapeDtypeStruct((OUT_ROWS, 128), jnp.float32),
        scratch_types=[
            pltpu.VMEM((EPW, F), jnp.int32),
            pltpu.VMEM((EPW, F), jnp.float32),
            pltpu.VMEM((RPW, D), jnp.float32),
            pltpu.VMEM((ORPW, 128), jnp.float32),
            pltpu.SemaphoreType.DMA,
        ],
    )
    def sc_pool(idx_hbm, fv_hbm, table_hbm, out_hbm,
                idx_v, fv_v, rows_v, bi_v, sem):
        wid = lax.axis_index("s") * 2 + lax.axis_index("c")
        base = wid * EPW
        pltpu.sync_copy(idx_hbm.at[pl.ds(base, EPW)], idx_v)
        pltpu.sync_copy(fv_hbm.at[pl.ds(base, EPW)], fv_v)

        def fire(e, carry):
            pltpu.async_copy(table_hbm.at[idx_v.at[e]],
                             rows_v.at[pl.ds(e * F, F)], sem)
            return carry
        lax.fori_loop(0, EPW, fire, 0)
        pltpu.make_async_copy(table_hbm.at[pl.ds(0, RPW)], rows_v, sem).wait()

        def e_body(e, carry):
            s = jnp.zeros((D,), jnp.float32)
            q = jnp.zeros((D,), jnp.float32)
            for f in range(F):
                row = rows_v[e * F + f]
                vb = plsc.load_gather(
                    fv_v, [jnp.full((D,), e, jnp.int32),
                           jnp.full((D,), f, jnp.int32)])
                ve = row * vb
                s = s + ve
                q = q + ve * ve
            bi = (s * s - q) * 0.5
            bi_v[e // PACK, pl.ds((e % PACK) * D, D)] = bi
            return carry

        lax.fori_loop(0, EPW, e_body, 0)
        pltpu.sync_copy(bi_v, out_hbm.at[pl.ds(wid * ORPW, ORPW)])

    return sc_pool


def _tc_mlp(bi_p, W1, b1, W2, b2, W3, b3):
    eye = jnp.eye(PACK, dtype=jnp.float32)
    W1p = jnp.kron(eye, W1)
    W2p = jnp.kron(eye, W2)
    W3p = jnp.kron(eye, W3)
    b1p = jnp.tile(b1, PACK).reshape(1, -1)
    b2p = jnp.tile(b2, PACK).reshape(1, -1)
    b3p = jnp.tile(b3, PACK).reshape(1, -1)

    def body(bi_ref, W1_ref, b1_ref, W2_ref, b2_ref, W3_ref, b3_ref, out_ref):
        h = jnp.maximum(
            jnp.dot(bi_ref[...], W1_ref[...], preferred_element_type=jnp.float32)
            + b1_ref[...], 0.0)
        h = jnp.maximum(
            jnp.dot(h, W2_ref[...], preferred_element_type=jnp.float32)
            + b2_ref[...], 0.0)
        o = jnp.dot(h, W3_ref[...], preferred_element_type=jnp.float32) + b3_ref[...]
        out_ref[...] = jax.nn.sigmoid(o)

    out = pl.pallas_call(
        body,
        out_shape=jax.ShapeDtypeStruct((OUT_ROWS, PACK), jnp.float32),
    )(bi_p, W1p, b1p, W2p, b2p, W3p, b3p)
    return out.reshape(B, 1)


def kernel(feat_index, feat_value, emb_table, W1, b1, W2, b2, W3, b3):
    fidx = feat_index.astype(jnp.int32)
    table_lin = _make_sc_detile()(emb_table.T, emb_table[V - 64:])\
        .reshape(V, D)
    bi_p = _make_sc_pool()(fidx, feat_value, table_lin)
    return _tc_mlp(bi_p, W1, b1, W2, b2, W3, b3)
